# baseline (speedup 1.0000x reference)
import functools

import jax
import jax.numpy as jnp
from jax import lax
from jax.experimental import pallas as pl
from jax.experimental.pallas import tpu as pltpu
from jax.experimental.pallas import tpu_sc as plsc

NUM_USERS = 1000000
NUM_ITEMS = 1000000
NUM_LABELS = 5
EMBED_DIM = 32
BATCH = 16384

_PACK = 4
_PACK_W = _PACK * EMBED_DIM

_INFO = plsc.get_sparse_core_info()
_NC, _NS, _LANES = _INFO.num_cores, _INFO.num_subcores, _INFO.num_lanes
_NW = _NC * _NS
_BPW = BATCH // _NW
_HALF = _BPW // 2
_IDX_CHUNK = 128
_NCHUNK = _HALF // _IDX_CHUNK
_NGROUP = _HALF // _LANES
_NB = NUM_LABELS - 1
_BROWS = (_HALF * _NB) // _IDX_CHUNK

_mesh = plsc.VectorSubcoreMesh(core_axis_name="c", subcore_axis_name="s")


@functools.partial(
    pl.kernel,
    mesh=_mesh,
    compiler_params=pltpu.CompilerParams(
        needs_layout_passes=False, use_tc_tiling_on_sc=True),
    out_type=jax.ShapeDtypeStruct((8, BATCH), jnp.float32),
    scratch_types=[
        pltpu.VMEM((_BPW,), jnp.int32),
        pltpu.VMEM((_BPW,), jnp.int32),
        pltpu.VMEM((_NCHUNK, _IDX_CHUNK), jnp.int32),
        pltpu.VMEM((_NCHUNK, _IDX_CHUNK), jnp.int32),
        pltpu.VMEM((_BROWS, _IDX_CHUNK), jnp.int32),
        pltpu.VMEM((_HALF, _PACK_W), jnp.float32),
        pltpu.VMEM((_HALF, _PACK_W), jnp.float32),
        pltpu.VMEM((_BROWS, _IDX_CHUNK), jnp.float32),
        pltpu.VMEM((8, _BPW), jnp.float32),
        pltpu.SemaphoreType.DMA,
    ],
)
def _ordrec_sc(uid_hbm, iid_hbm, uemb_hbm, iemb_hbm, ubeta_hbm, out_hbm,
               uid_v, iid_v, urow_v, irow_v, bel_v, u_rows, i_rows, b_rows,
               out_v, sem):
    wid = lax.axis_index("s") * _NC + lax.axis_index("c")
    base = wid * _BPW

    pltpu.sync_copy(uid_hbm.at[pl.ds(base, _BPW)], uid_v)
    pltpu.sync_copy(iid_hbm.at[pl.ds(base, _BPW)], iid_v)

    lanes = lax.iota(jnp.int32, _LANES)

    for half in range(2):
        hbase = half * _HALF
        for j in range(_NCHUNK):
            for k in range(_IDX_CHUNK // _LANES):
                sl = pl.ds(hbase + j * _IDX_CHUNK + k * _LANES, _LANES)
                dsl = pl.ds(k * _LANES, _LANES)
                urow_v[j, dsl] = lax.shift_right_logical(uid_v[sl], 2)
                irow_v[j, dsl] = lax.shift_right_logical(iid_v[sl], 2)
        for jj in range(_NB):
            for r in range(_BROWS // _NB):
                for k in range(_IDX_CHUNK // _LANES):
                    sl = pl.ds(hbase + r * _IDX_CHUNK + k * _LANES, _LANES)
                    row = jj * (_BROWS // _NB) + r
                    bel_v[row, pl.ds(k * _LANES, _LANES)] = (
                        uid_v[sl] + jj * NUM_USERS)

        copies = []
        for j in range(_NCHUNK):
            sl = pl.ds(j * _IDX_CHUNK, _IDX_CHUNK)
            copies.append(pltpu.async_copy(uemb_hbm.at[urow_v.at[j]],
                                           u_rows.at[sl], sem))
            copies.append(pltpu.async_copy(iemb_hbm.at[irow_v.at[j]],
                                           i_rows.at[sl], sem))
        for r in range(_BROWS):
            copies.append(pltpu.async_copy(ubeta_hbm.at[bel_v.at[r]],
                                           b_rows.at[r], sem))
        for c in copies:
            c.wait()

        def group(g, carry):
            rows = g * _LANES + lanes
            uidv = plsc.load_gather(uid_v, [hbase + rows])
            iidv = plsc.load_gather(iid_v, [hbase + rows])
            uoff = lax.shift_left(uidv & (_PACK - 1), 5)
            ioff = lax.shift_left(iidv & (_PACK - 1), 5)
            y = jnp.zeros((_LANES,), jnp.float32)
            for d in range(EMBED_DIM):
                y = y + (plsc.load_gather(u_rows, [rows, uoff + d]) *
                         plsc.load_gather(i_rows, [rows, ioff + d]))

            def beta(j):
                pos = j * _HALF + g * _LANES
                return plsc.load_gather(
                    b_rows,
                    [jnp.full((_LANES,), pos // _IDX_CHUNK, jnp.int32),
                     pos % _IDX_CHUNK + lanes])

            out_rows = hbase + rows
            cum = beta(0)
            s_prev = 1.0 / (1.0 + jnp.exp(y - cum))
            plsc.store_scatter(out_v,
                               [jnp.zeros((_LANES,), jnp.int32), out_rows],
                               s_prev)
            for j in range(1, _NB):
                cum = cum + jnp.exp(beta(j))
                s = 1.0 / (1.0 + jnp.exp(y - cum))
                plsc.store_scatter(out_v,
                                   [jnp.full((_LANES,), j, jnp.int32),
                                    out_rows],
                                   s - s_prev)
                s_prev = s
            plsc.store_scatter(out_v,
                               [jnp.full((_LANES,), _NB, jnp.int32),
                                out_rows],
                               1.0 - s_prev)
            for j in range(NUM_LABELS, 8):
                plsc.store_scatter(out_v,
                                   [jnp.full((_LANES,), j, jnp.int32),
                                    out_rows],
                                   jnp.zeros((_LANES,), jnp.float32))
            return carry

        lax.fori_loop(0, _NGROUP, group, jnp.int32(0))

    pltpu.sync_copy(out_v, out_hbm.at[:, pl.ds(base, _BPW)])


def kernel(user_ids, item_ids, user_embeddings, item_embeddings, user_betas):
    def _pack_rows(t):
        return jnp.concatenate([t[m::_PACK] for m in range(_PACK)], axis=1)

    packed_u = _pack_rows(user_embeddings)
    packed_i = _pack_rows(item_embeddings)
    flat_b = user_betas.T.reshape(-1)
    out8 = _ordrec_sc(user_ids, item_ids, packed_u, packed_i, flat_b)
    return out8[:NUM_LABELS].T


# speedup vs baseline: 9.7308x; 9.7308x over previous
import functools

import jax
import jax.numpy as jnp
from jax import lax
from jax.experimental import pallas as pl
from jax.experimental.pallas import tpu as pltpu
from jax.experimental.pallas import tpu_sc as plsc

NUM_USERS = 1000000
NUM_ITEMS = 1000000
NUM_LABELS = 5
EMBED_DIM = 32
BATCH = 16384

_PACK = 4
_PACK_W = _PACK * EMBED_DIM

_INFO = plsc.get_sparse_core_info()
_NC, _NS, _LANES = _INFO.num_cores, _INFO.num_subcores, _INFO.num_lanes
_NW = _NC * _NS
_BPW = BATCH // _NW
_HALF = _BPW // 2
_IDX_CHUNK = 128
_NCHUNK = _HALF // _IDX_CHUNK
_NGROUP = _HALF // _LANES
_NB = NUM_LABELS - 1
_BROWS = (_HALF * _NB) // _IDX_CHUNK

_mesh = plsc.VectorSubcoreMesh(core_axis_name="c", subcore_axis_name="s")


@functools.partial(
    pl.kernel,
    mesh=_mesh,
    compiler_params=pltpu.CompilerParams(
        needs_layout_passes=False, use_tc_tiling_on_sc=True),
    out_type=jax.ShapeDtypeStruct((8, BATCH), jnp.float32),
    scratch_types=[
        pltpu.VMEM((_BPW,), jnp.int32),
        pltpu.VMEM((_BPW,), jnp.int32),
        pltpu.VMEM((_NCHUNK, _IDX_CHUNK), jnp.int32),
        pltpu.VMEM((_NCHUNK, _IDX_CHUNK), jnp.int32),
        pltpu.VMEM((_BROWS, _IDX_CHUNK), jnp.int32),
        pltpu.VMEM((_HALF, _PACK_W), jnp.float32),
        pltpu.VMEM((_HALF, _PACK_W), jnp.float32),
        pltpu.VMEM((_BROWS, _IDX_CHUNK), jnp.float32),
        pltpu.VMEM((8, _BPW), jnp.float32),
        pltpu.SemaphoreType.DMA,
    ],
)
def _ordrec_sc(uid_hbm, iid_hbm, uemb_hbm, iemb_hbm, ubeta_hbm, out_hbm,
               uid_v, iid_v, urow_v, irow_v, bel_v, u_rows, i_rows, b_rows,
               out_v, sem):
    wid = lax.axis_index("s") * _NC + lax.axis_index("c")
    base = wid * _BPW

    pltpu.sync_copy(uid_hbm.at[pl.ds(base, _BPW)], uid_v)
    pltpu.sync_copy(iid_hbm.at[pl.ds(base, _BPW)], iid_v)

    lanes = lax.iota(jnp.int32, _LANES)

    for half in range(2):
        hbase = half * _HALF
        for j in range(_NCHUNK):
            for k in range(_IDX_CHUNK // _LANES):
                sl = pl.ds(hbase + j * _IDX_CHUNK + k * _LANES, _LANES)
                dsl = pl.ds(k * _LANES, _LANES)
                urow_v[j, dsl] = lax.shift_right_logical(uid_v[sl], 2)
                irow_v[j, dsl] = lax.shift_right_logical(iid_v[sl], 2)
        for jj in range(_NB):
            for r in range(_BROWS // _NB):
                for k in range(_IDX_CHUNK // _LANES):
                    sl = pl.ds(hbase + r * _IDX_CHUNK + k * _LANES, _LANES)
                    row = jj * (_BROWS // _NB) + r
                    bel_v[row, pl.ds(k * _LANES, _LANES)] = (
                        uid_v[sl] + jj * NUM_USERS)

        copies = []
        for j in range(_NCHUNK):
            sl = pl.ds(j * _IDX_CHUNK, _IDX_CHUNK)
            copies.append(pltpu.async_copy(uemb_hbm.at[urow_v.at[j]],
                                           u_rows.at[sl], sem))
            copies.append(pltpu.async_copy(iemb_hbm.at[irow_v.at[j]],
                                           i_rows.at[sl], sem))
        for r in range(_BROWS):
            copies.append(pltpu.async_copy(ubeta_hbm.at[bel_v.at[r]],
                                           b_rows.at[r], sem))
        for c in copies:
            c.wait()

        def group(g, carry):
            rows = g * _LANES + lanes
            uidv = plsc.load_gather(uid_v, [hbase + rows])
            iidv = plsc.load_gather(iid_v, [hbase + rows])
            uoff = lax.shift_left(uidv & (_PACK - 1), 5)
            ioff = lax.shift_left(iidv & (_PACK - 1), 5)
            y = jnp.zeros((_LANES,), jnp.float32)
            for d in range(EMBED_DIM):
                y = y + (plsc.load_gather(u_rows, [rows, uoff + d]) *
                         plsc.load_gather(i_rows, [rows, ioff + d]))

            def beta(j):
                pos = j * _HALF + g * _LANES
                return plsc.load_gather(
                    b_rows,
                    [jnp.full((_LANES,), pos // _IDX_CHUNK, jnp.int32),
                     pos % _IDX_CHUNK + lanes])

            out_rows = hbase + rows
            cum = beta(0)
            s_prev = 1.0 / (1.0 + jnp.exp(y - cum))
            plsc.store_scatter(out_v,
                               [jnp.zeros((_LANES,), jnp.int32), out_rows],
                               s_prev)
            for j in range(1, _NB):
                cum = cum + jnp.exp(beta(j))
                s = 1.0 / (1.0 + jnp.exp(y - cum))
                plsc.store_scatter(out_v,
                                   [jnp.full((_LANES,), j, jnp.int32),
                                    out_rows],
                                   s - s_prev)
                s_prev = s
            plsc.store_scatter(out_v,
                               [jnp.full((_LANES,), _NB, jnp.int32),
                                out_rows],
                               1.0 - s_prev)
            for j in range(NUM_LABELS, 8):
                plsc.store_scatter(out_v,
                                   [jnp.full((_LANES,), j, jnp.int32),
                                    out_rows],
                                   jnp.zeros((_LANES,), jnp.float32))
            return carry

        lax.fori_loop(0, _NGROUP, group, jnp.int32(0))

    pltpu.sync_copy(out_v, out_hbm.at[:, pl.ds(base, _BPW)])


def kernel(user_ids, item_ids, user_embeddings, item_embeddings, user_betas):
    packed_u, packed_i = jax.lax.optimization_barrier(
        (user_embeddings.reshape(NUM_USERS // _PACK, _PACK_W),
         item_embeddings.reshape(NUM_ITEMS // _PACK, _PACK_W)))
    flat_b = user_betas.T.reshape(-1)
    out8 = _ordrec_sc(user_ids, item_ids, packed_u, packed_i, flat_b)
    return out8[:NUM_LABELS].T
